# baseline (speedup 1.0000x reference)
import functools

import jax
import jax.numpy as jnp
from jax import lax
from jax.experimental import pallas as pl
from jax.experimental.pallas import tpu as pltpu
from jax.experimental.pallas import tpu_sc as plsc

K = 10
N_WAY = 5
NC = 16
D = 256
S = 4096
Q = 2048
QT = 256
SEGW = 16
NSEG = S // SEGW
POWER = 2
DELTA = 1.0

_NW = 32
_RPW = Q // _NW


def _stage_a(support_ref, w_ref, labels_ref, zs_ref, proto_ref, s2_ref):
    zs = jnp.dot(support_ref[...], w_ref[...], preferred_element_type=jnp.float32)
    zs_ref[...] = zs
    labels = labels_ref[...]
    classes = jax.lax.broadcasted_iota(jnp.int32, (NC, S), 0)
    onehot_t = (labels == classes).astype(jnp.float32)
    counts = jnp.sum(onehot_t, axis=1, keepdims=True)
    proto_sums = jnp.dot(onehot_t, zs, preferred_element_type=jnp.float32)
    proto_ref[...] = proto_sums / jnp.maximum(counts, 1.0)
    s2_ref[...] = jnp.sum(zs * zs, axis=1, keepdims=True).reshape(1, S)


def _stage_b(q_ref, w_ref, zs_ref, labels_ref, proto_ref, s2_ref,
             packed_ref, seg_ref, dp_ref):
    zq = jnp.dot(q_ref[...], w_ref[...], preferred_element_type=jnp.float32)
    q2 = jnp.sum(zq * zq, axis=1, keepdims=True)
    cross = jax.lax.dot_general(zq, zs_ref[...], (((1,), (1,)), ((), ())),
                                preferred_element_type=jnp.float32)
    d2 = jnp.maximum(q2 + s2_ref[...] - 2.0 * cross, 0.0)

    bits = jax.lax.bitcast_convert_type(d2, jnp.int32)
    packed = (bits & ~jnp.int32(7)) | labels_ref[...]
    packed_ref[...] = packed

    segmin = packed[:, :NSEG]
    for w in range(1, SEGW):
        segmin = jnp.minimum(segmin, packed[:, w * NSEG:(w + 1) * NSEG])
    segid = jax.lax.broadcasted_iota(jnp.int32, (QT, NSEG), 1)
    seg_ref[...] = (segmin & ~jnp.int32(0xFF)) | segid

    proto = proto_ref[...]
    p2 = jnp.sum(proto * proto, axis=1, keepdims=True).reshape(1, NC)
    crossp = jax.lax.dot_general(zq, proto, (((1,), (1,)), ((), ())),
                                 preferred_element_type=jnp.float32)
    dp2 = q2 + p2 - 2.0 * crossp
    dp_ref[...] = jnp.sqrt(jnp.maximum(dp2, 1e-12))


def _sort16(x):
    return plsc.sort_key_val(x, x)[0]


def _merge16(a, b):
    lo = jnp.minimum(a, lax.rev(b, (0,)))
    return _sort16(lo)


def _tournament(vecs):
    vecs = [_sort16(v) for v in vecs]
    while len(vecs) > 1:
        vecs = [_merge16(vecs[i], vecs[i + 1]) for i in range(0, len(vecs), 2)]
    return vecs[0]


_CH = 8


def _stage_c_body(seg_hbm, packed_hbm, dp_hbm, out_hbm,
                  seg_v, dp_v, out_v, rows_v, cnt_v, sem, dsem, *, rpw):
    nchunk = rpw // _CH
    wid = lax.axis_index("s") * 2 + lax.axis_index("c")
    base = wid * rpw

    pltpu.async_copy(seg_hbm.at[pl.ds(base, rpw)], seg_v, sem).wait()
    pltpu.async_copy(dp_hbm.at[pl.ds(base, rpw)], dp_v, dsem).wait()

    lane = lax.iota(jnp.int32, 16)
    ones = jnp.ones((16,), jnp.float32)
    topmask = lane < K

    handles = [None, None]
    sems = [sem, dsem]
    handles[0] = pltpu.async_copy(
        packed_hbm.at[pl.ds(base, _CH)], rows_v.at[0], sems[0])
    for c in range(nchunk):
        buf = c % 2
        handles[buf].wait()
        if c + 1 < nchunk:
            nbuf = (c + 1) % 2
            handles[nbuf] = pltpu.async_copy(
                packed_hbm.at[pl.ds(base + (c + 1) * _CH, _CH)],
                rows_v.at[nbuf], sems[nbuf])
        buf_splat = jnp.full((16,), buf, jnp.int32)

        def row_body(rl, carry, c=c, buf_splat=buf_splat):
            r = c * _CH + rl
            r_splat = jnp.full((16,), r, jnp.int32)
            rl_splat = jnp.full((16,), rl, jnp.int32)

            segs = [plsc.load_gather(seg_v, [r_splat, lane + 16 * j])
                    for j in range(16)]
            best_seg = _tournament(segs)

            col0 = best_seg & 0xFF
            wins = [plsc.load_gather(rows_v, [buf_splat, rl_splat, col0 + p * NSEG])
                    for p in range(SEGW)]
            best = _tournament(wins)

            cnt_v[...] = jnp.zeros((16,), jnp.float32)
            plsc.addupdate_scatter(cnt_v, [best & 7], ones, mask=topmask)
            cnt = cnt_v[...]
            dp = plsc.load_gather(dp_v, [r_splat, lane])
            plsc.store_scatter(out_v, [r_splat, lane],
                               -(dp / (cnt * cnt + DELTA)))
            return carry

        lax.fori_loop(0, _CH, row_body, None)

    pltpu.async_copy(out_v, out_hbm.at[pl.ds(base, rpw)], sem).wait()


_NSPLIT = 2
_QH = Q // _NSPLIT


def _run_stage_c(seg, packed, dp, nrows):
    rpw = nrows // _NW
    mesh = plsc.VectorSubcoreMesh(core_axis_name="c", subcore_axis_name="s")
    return pl.kernel(
        functools.partial(_stage_c_body, rpw=rpw),
        mesh=mesh,
        compiler_params=pltpu.CompilerParams(needs_layout_passes=False),
        out_type=jax.ShapeDtypeStruct((nrows, NC), jnp.float32),
        scratch_types=[
            pltpu.VMEM((rpw, NSEG), jnp.int32),
            pltpu.VMEM((rpw, NC), jnp.float32),
            pltpu.VMEM((rpw, NC), jnp.float32),
            pltpu.VMEM((2, _CH, S), jnp.int32),
            pltpu.VMEM((16,), jnp.float32),
            pltpu.SemaphoreType.DMA,
            pltpu.SemaphoreType.DMA,
        ],
    )(seg, packed, dp)


@jax.jit
def kernel(support_images, support_labels, query_images, W):
    labels2d = support_labels.reshape(1, S)

    zs, proto, s2 = pl.pallas_call(
        _stage_a,
        out_shape=[
            jax.ShapeDtypeStruct((S, D), jnp.float32),
            jax.ShapeDtypeStruct((NC, D), jnp.float32),
            jax.ShapeDtypeStruct((1, S), jnp.float32),
        ],
    )(support_images, W, labels2d)

    grid = _QH // QT
    outs = []
    for h in range(_NSPLIT):
        qh = jax.lax.slice_in_dim(query_images, h * _QH, (h + 1) * _QH)
        packed, seg, dp = pl.pallas_call(
            _stage_b,
            grid=(grid,),
            in_specs=[
                pl.BlockSpec((QT, D), lambda i: (i, 0)),
                pl.BlockSpec((D, D), lambda i: (0, 0)),
                pl.BlockSpec((S, D), lambda i: (0, 0)),
                pl.BlockSpec((1, S), lambda i: (0, 0)),
                pl.BlockSpec((NC, D), lambda i: (0, 0)),
                pl.BlockSpec((1, S), lambda i: (0, 0)),
            ],
            out_specs=[
                pl.BlockSpec((QT, S), lambda i: (i, 0)),
                pl.BlockSpec((QT, NSEG), lambda i: (i, 0)),
                pl.BlockSpec((QT, NC), lambda i: (i, 0)),
            ],
            out_shape=[
                jax.ShapeDtypeStruct((_QH, S), jnp.int32),
                jax.ShapeDtypeStruct((_QH, NSEG), jnp.int32),
                jax.ShapeDtypeStruct((_QH, NC), jnp.float32),
            ],
        )(qh, W, zs, labels2d, proto, s2)
        outs.append(_run_stage_c(seg, packed, dp, _QH))

    scores = jnp.concatenate(outs, axis=0)
    return scores[:, :N_WAY]


# speedup vs baseline: 1.1227x; 1.1227x over previous
import functools

import jax
import jax.numpy as jnp
from jax import lax
from jax.experimental import pallas as pl
from jax.experimental.pallas import tpu as pltpu
from jax.experimental.pallas import tpu_sc as plsc

K = 10
N_WAY = 5
NC = 16
D = 256
S = 4096
Q = 2048
QT = 512
SEGW = 16
NSEG = S // SEGW
POWER = 2
DELTA = 1.0

_NW = 32
_RPW = Q // _NW


def _stage_a(support_ref, w_ref, labels_ref, zs_ref, proto_ref, s2_ref):
    zs = jnp.dot(support_ref[...], w_ref[...], preferred_element_type=jnp.float32)
    zs_ref[...] = zs
    labels = labels_ref[...]
    classes = jax.lax.broadcasted_iota(jnp.int32, (NC, S), 0)
    onehot_t = (labels == classes).astype(jnp.float32)
    counts = jnp.sum(onehot_t, axis=1, keepdims=True)
    proto_sums = jnp.dot(onehot_t, zs, preferred_element_type=jnp.float32)
    proto_ref[...] = proto_sums / jnp.maximum(counts, 1.0)
    s2_ref[...] = jnp.sum(zs * zs, axis=1, keepdims=True).reshape(1, S)


def _stage_b(q_ref, w_ref, zs_ref, labels_ref, proto_ref, s2_ref,
             packed_ref, seg_ref, dp_ref):
    zq = jnp.dot(q_ref[...], w_ref[...], preferred_element_type=jnp.float32)
    q2 = jnp.sum(zq * zq, axis=1, keepdims=True)
    cross = jax.lax.dot_general(zq, zs_ref[...], (((1,), (1,)), ((), ())),
                                preferred_element_type=jnp.float32)
    d2 = jnp.maximum(q2 + s2_ref[...] - 2.0 * cross, 0.0)

    bits = jax.lax.bitcast_convert_type(d2, jnp.int32)
    packed = (bits & ~jnp.int32(7)) | labels_ref[...]
    packed_ref[...] = packed

    segmin = packed[:, :NSEG]
    for w in range(1, SEGW):
        segmin = jnp.minimum(segmin, packed[:, w * NSEG:(w + 1) * NSEG])
    segid = jax.lax.broadcasted_iota(jnp.int32, (QT, NSEG), 1)
    seg_ref[...] = (segmin & ~jnp.int32(0xFF)) | segid

    proto = proto_ref[...]
    p2 = jnp.sum(proto * proto, axis=1, keepdims=True).reshape(1, NC)
    crossp = jax.lax.dot_general(zq, proto, (((1,), (1,)), ((), ())),
                                 preferred_element_type=jnp.float32)
    dp2 = q2 + p2 - 2.0 * crossp
    dp_ref[...] = jnp.sqrt(jnp.maximum(dp2, 1e-12))


def _sort16(x):
    return plsc.sort_key_val(x, x)[0]


def _merge16(a, b):
    lo = jnp.minimum(a, lax.rev(b, (0,)))
    return _sort16(lo)


def _tournament(vecs):
    vecs = [_sort16(v) for v in vecs]
    while len(vecs) > 1:
        vecs = [_merge16(vecs[i], vecs[i + 1]) for i in range(0, len(vecs), 2)]
    return vecs[0]


_CH = 8


def _stage_c_body(seg_hbm, packed_hbm, dp_hbm, out_hbm,
                  seg_v, dp_v, out_v, rows_v, cnt_v, sem, dsem, *, rpw):
    nchunk = rpw // _CH
    wid = lax.axis_index("s") * 2 + lax.axis_index("c")
    base = wid * rpw

    pltpu.async_copy(seg_hbm.at[pl.ds(base, rpw)], seg_v, sem).wait()
    pltpu.async_copy(dp_hbm.at[pl.ds(base, rpw)], dp_v, dsem).wait()

    lane = lax.iota(jnp.int32, 16)
    ones = jnp.ones((16,), jnp.float32)
    topmask = lane < K

    handles = [None, None]
    sems = [sem, dsem]
    handles[0] = pltpu.async_copy(
        packed_hbm.at[pl.ds(base, _CH)], rows_v.at[0], sems[0])
    for c in range(nchunk):
        buf = c % 2
        handles[buf].wait()
        if c + 1 < nchunk:
            nbuf = (c + 1) % 2
            handles[nbuf] = pltpu.async_copy(
                packed_hbm.at[pl.ds(base + (c + 1) * _CH, _CH)],
                rows_v.at[nbuf], sems[nbuf])
        buf_splat = jnp.full((16,), buf, jnp.int32)

        def row_body(rl, carry, c=c, buf_splat=buf_splat):
            r = c * _CH + rl
            r_splat = jnp.full((16,), r, jnp.int32)
            rl_splat = jnp.full((16,), rl, jnp.int32)

            segs = [plsc.load_gather(seg_v, [r_splat, lane + 16 * j])
                    for j in range(16)]
            best_seg = _tournament(segs)

            col0 = best_seg & 0xFF
            wins = [plsc.load_gather(rows_v, [buf_splat, rl_splat, col0 + p * NSEG])
                    for p in range(SEGW)]
            best = _tournament(wins)

            cnt_v[...] = jnp.zeros((16,), jnp.float32)
            plsc.addupdate_scatter(cnt_v, [best & 7], ones, mask=topmask)
            cnt = cnt_v[...]
            dp = plsc.load_gather(dp_v, [r_splat, lane])
            plsc.store_scatter(out_v, [r_splat, lane],
                               -(dp / (cnt * cnt + DELTA)))
            return carry

        lax.fori_loop(0, _CH, row_body, None)

    pltpu.async_copy(out_v, out_hbm.at[pl.ds(base, rpw)], sem).wait()


_NSPLIT = 1
_QH = Q // _NSPLIT


def _run_stage_c(seg, packed, dp, nrows):
    rpw = nrows // _NW
    mesh = plsc.VectorSubcoreMesh(core_axis_name="c", subcore_axis_name="s")
    return pl.kernel(
        functools.partial(_stage_c_body, rpw=rpw),
        mesh=mesh,
        compiler_params=pltpu.CompilerParams(needs_layout_passes=False),
        out_type=jax.ShapeDtypeStruct((nrows, NC), jnp.float32),
        scratch_types=[
            pltpu.VMEM((rpw, NSEG), jnp.int32),
            pltpu.VMEM((rpw, NC), jnp.float32),
            pltpu.VMEM((rpw, NC), jnp.float32),
            pltpu.VMEM((2, _CH, S), jnp.int32),
            pltpu.VMEM((16,), jnp.float32),
            pltpu.SemaphoreType.DMA,
            pltpu.SemaphoreType.DMA,
        ],
    )(seg, packed, dp)


@jax.jit
def kernel(support_images, support_labels, query_images, W):
    labels2d = support_labels.reshape(1, S)

    zs, proto, s2 = pl.pallas_call(
        _stage_a,
        out_shape=[
            jax.ShapeDtypeStruct((S, D), jnp.float32),
            jax.ShapeDtypeStruct((NC, D), jnp.float32),
            jax.ShapeDtypeStruct((1, S), jnp.float32),
        ],
    )(support_images, W, labels2d)

    grid = _QH // QT
    outs = []
    for h in range(_NSPLIT):
        qh = jax.lax.slice_in_dim(query_images, h * _QH, (h + 1) * _QH)
        packed, seg, dp = pl.pallas_call(
            _stage_b,
            grid=(grid,),
            in_specs=[
                pl.BlockSpec((QT, D), lambda i: (i, 0)),
                pl.BlockSpec((D, D), lambda i: (0, 0)),
                pl.BlockSpec((S, D), lambda i: (0, 0)),
                pl.BlockSpec((1, S), lambda i: (0, 0)),
                pl.BlockSpec((NC, D), lambda i: (0, 0)),
                pl.BlockSpec((1, S), lambda i: (0, 0)),
            ],
            out_specs=[
                pl.BlockSpec((QT, S), lambda i: (i, 0)),
                pl.BlockSpec((QT, NSEG), lambda i: (i, 0)),
                pl.BlockSpec((QT, NC), lambda i: (i, 0)),
            ],
            out_shape=[
                jax.ShapeDtypeStruct((_QH, S), jnp.int32),
                jax.ShapeDtypeStruct((_QH, NSEG), jnp.int32),
                jax.ShapeDtypeStruct((_QH, NC), jnp.float32),
            ],
        )(qh, W, zs, labels2d, proto, s2)
        outs.append(_run_stage_c(seg, packed, dp, _QH))

    scores = jnp.concatenate(outs, axis=0)
    return scores[:, :N_WAY]
